# speedup vs baseline: 1.5939x; 1.5939x over previous
import jax
import jax.numpy as jnp
from jax.experimental import pallas as pl

_A = 3
_ATTR = 85
_G = 76
_S = _G * _G
_STRIDE = 8.0
_ANCH_W = (116.0, 156.0, 373.0)
_ANCH_H = (90.0, 198.0, 326.0)


def _decode_kernel(x_ref, o_ref):
    a = pl.program_id(1)
    x = x_ref[0, 0]

    row = jax.lax.broadcasted_iota(jnp.int32, x.shape, 0)
    col = jax.lax.broadcasted_iota(jnp.int32, x.shape, 1)

    is_w = row == 2
    is_h = row == 3
    val = jnp.where(is_w | is_h, jnp.exp(x), jax.nn.sigmoid(x))

    gx = (col % _G).astype(jnp.float32)
    gy = (col // _G).astype(jnp.float32)
    add = jnp.where(row == 0, gx, jnp.where(row == 1, gy, 0.0))

    aw = jnp.where(a == 0, _ANCH_W[0], jnp.where(a == 1, _ANCH_W[1], _ANCH_W[2]))
    ah = jnp.where(a == 0, _ANCH_H[0], jnp.where(a == 1, _ANCH_H[1], _ANCH_H[2]))
    mult = jnp.where(row < 2, _STRIDE, jnp.where(is_w, aw, jnp.where(is_h, ah, 1.0)))

    y = (val + add) * mult
    o_ref[0, 0] = y.T


def kernel(inputs):
    b = inputs.shape[0]
    x = inputs.reshape(b, _A, _ATTR, _S)
    out = pl.pallas_call(
        _decode_kernel,
        grid=(b, _A),
        in_specs=[pl.BlockSpec((1, 1, _ATTR, _S), lambda i, j: (i, j, 0, 0))],
        out_specs=pl.BlockSpec((1, 1, _S, _ATTR), lambda i, j: (i, j, 0, 0)),
        out_shape=jax.ShapeDtypeStruct((b, _A, _S, _ATTR), jnp.float32),
    )(x)
    return out.reshape(b, _A * _S, _ATTR)


# speedup vs baseline: 1.5942x; 1.0002x over previous
import jax
import jax.numpy as jnp
from jax.experimental import pallas as pl

_A = 3
_ATTR = 85
_G = 76
_S = _G * _G
_STRIDE = 8.0
_ANCH_W = (116.0, 156.0, 373.0)
_ANCH_H = (90.0, 198.0, 326.0)


def _decode_kernel(x_ref, o_ref):
    a = pl.program_id(1)
    x = x_ref[0, 0]

    sig = 0.5 * jnp.tanh(0.5 * x) + 0.5

    row2 = jax.lax.broadcasted_iota(jnp.int32, (2, _S), 0)
    col2 = jax.lax.broadcasted_iota(jnp.int32, (2, _S), 1)
    add = jnp.where(row2 == 0, col2 % _G, col2 // _G).astype(jnp.float32)
    top = (sig[0:2] + add) * _STRIDE

    aw = jnp.where(a == 0, _ANCH_W[0], jnp.where(a == 1, _ANCH_W[1], _ANCH_W[2]))
    ah = jnp.where(a == 0, _ANCH_H[0], jnp.where(a == 1, _ANCH_H[1], _ANCH_H[2]))
    mult = jnp.where(row2 == 0, aw, ah)
    mid = jnp.exp(x[2:4]) * mult

    y = jnp.concatenate([top, mid, sig[4:]], axis=0)
    o_ref[0, 0] = y.T


def kernel(inputs):
    b = inputs.shape[0]
    x = inputs.reshape(b, _A, _ATTR, _S)
    out = pl.pallas_call(
        _decode_kernel,
        grid=(b, _A),
        in_specs=[pl.BlockSpec((1, 1, _ATTR, _S), lambda i, j: (i, j, 0, 0))],
        out_specs=pl.BlockSpec((1, 1, _S, _ATTR), lambda i, j: (i, j, 0, 0)),
        out_shape=jax.ShapeDtypeStruct((b, _A, _S, _ATTR), jnp.float32),
    )(x)
    return out.reshape(b, _A * _S, _ATTR)
